# Initial kernel scaffold; baseline (speedup 1.0000x reference)
import jax
import jax.numpy as jnp
from jax.experimental import pallas as pl


def kernel(items, memories_h, memories_r, memories_t, neighbor_entities, neighbor_relations, entity_table, relation_table, relation_table_gcn, W_t0, b_t0, W_t1, b_t1, W_g0, b_g0):
    raise NotImplementedError("write your pallas kernel here")


# speedup vs baseline: 1.3892x; 1.3892x over previous
import functools

import jax
import jax.numpy as jnp
from jax import lax
from jax.experimental import pallas as pl
from jax.experimental.pallas import tpu as pltpu
from jax.experimental.pallas import tpu_sc as plsc

B = 1024
DIM = 32
N_MEM = 32
N_HOP = 2
N_NEI = 16
N_REL = 32

NC, NS = 2, 16
NW = NC * NS
CHUNK = 128

MB = 128
GRID = B // MB

SEG_ITEMS = 4096
N_MEMIDX = B * N_MEM
N_NBR = B * N_NEI
N_TOT = SEG_ITEMS + 2 * N_HOP * N_MEMIDX + N_NBR

def _sc_mesh():
    return plsc.VectorSubcoreMesh(core_axis_name="c", subcore_axis_name="s",
                                  num_cores=NC, num_subcores=NS)


def _nbr_gather(ne, nr, items):
    per_w = B // NW

    @functools.partial(
        pl.kernel,
        out_type=(jax.ShapeDtypeStruct((B, N_NEI), jnp.int32),
                  jax.ShapeDtypeStruct((B, N_NEI), jnp.int32)),
        mesh=_sc_mesh(),
        compiler_params=pltpu.CompilerParams(use_tc_tiling_on_sc=False),
        scratch_types=[pltpu.VMEM((per_w,), jnp.int32),
                       pltpu.VMEM((per_w, N_NEI), jnp.int32),
                       pltpu.SemaphoreType.DMA],
    )
    def k(ne_hbm, nr_hbm, items_hbm, ne_out, nr_out, idx_v, rows_v, sem):
        wid = lax.axis_index("s") * NC + lax.axis_index("c")
        base = wid * per_w
        pltpu.sync_copy(items_hbm.at[pl.ds(base, per_w)], idx_v)
        pltpu.async_copy(ne_hbm.at[idx_v], rows_v, sem).wait()
        pltpu.sync_copy(rows_v, ne_out.at[pl.ds(base, per_w), :])
        pltpu.async_copy(nr_hbm.at[idx_v], rows_v, sem).wait()
        pltpu.sync_copy(rows_v, nr_out.at[pl.ds(base, per_w), :])

    return k(ne, nr, items)


def _entity_gather(table, idx_all):
    n = idx_all.shape[0]
    per_w = n // NW
    n_chunks = per_w // CHUNK

    @functools.partial(
        pl.kernel,
        out_type=jax.ShapeDtypeStruct((n, DIM), jnp.float32),
        mesh=_sc_mesh(),
        compiler_params=pltpu.CompilerParams(use_tc_tiling_on_sc=False),
        scratch_types=[pltpu.VMEM((CHUNK,), jnp.int32),
                       pltpu.VMEM((CHUNK, DIM), jnp.float32),
                       pltpu.SemaphoreType.DMA],
    )
    def k(table_hbm, idx_hbm, out_hbm, idx_v, rows_v, sem):
        wid = lax.axis_index("s") * NC + lax.axis_index("c")
        base = wid * per_w

        def body(c, carry):
            off = base + c * CHUNK
            pltpu.sync_copy(idx_hbm.at[pl.ds(off, CHUNK)], idx_v)
            pltpu.async_copy(table_hbm.at[idx_v], rows_v, sem).wait()
            pltpu.sync_copy(rows_v, out_hbm.at[pl.ds(off, CHUNK), :])
            return carry

        lax.fori_loop(0, n_chunks, body, 0)

    return k(table, idx_all)


def _tc_body(g_it, g_h0, g_h1, g_t0, g_t1, g_nbr, mr_ref, nr_ref,
             R3_ref, Rm_ref, rtg_ref, Wt0_ref, Wt1_ref, Wg0_ref,
             bt0_ref, bt1_ref, bg0_ref, preds_ref, kge_ref):
    i = pl.program_id(0)
    f32 = jnp.float32
    v0 = g_it[...]
    v = v0
    R3 = R3_ref[...]
    Rmean = Rm_ref[...]
    kge_acc = f32(0.0)
    hs = (g_h0, g_h1)
    ts = (g_t0, g_t1)
    Ws = (Wt0_ref, Wt1_ref)
    bs = (bt0_ref, bt1_ref)
    for hop in range(N_HOP):
        h = hs[hop][...].reshape(MB, N_MEM, DIM)
        t = ts[hop][...].reshape(MB, N_MEM, DIM)
        r = mr_ref[hop]
        u = jnp.dot(v, R3, preferred_element_type=f32)
        att_s = jnp.zeros((MB, N_MEM), f32)
        for rel in range(N_REL):
            oh = (r == rel).astype(f32)
            urel = u[:, rel * DIM:(rel + 1) * DIM]
            hdotu = jnp.sum(h * urel[:, None, :], axis=2)
            att_s = att_s + oh * hdotu
        att_s = att_s - jnp.max(att_s, axis=1, keepdims=True)
        e = jnp.exp(att_s)
        att = e / jnp.sum(e, axis=1, keepdims=True)
        o = jnp.sum(att[:, :, None] * t, axis=1)
        oh3 = (r[:, :, None] ==
               lax.broadcasted_iota(jnp.int32, (MB, N_MEM, N_REL), 2)).astype(f32)
        Rmsel = jnp.dot(oh3.reshape(MB * N_MEM, N_REL), Rmean,
                        preferred_element_type=f32).reshape(MB, N_MEM, DIM)
        diff = h - t + Rmsel
        kge_acc = kge_acc + jnp.sum(diff * diff)
        v = jnp.tanh(jnp.dot(o + v, Ws[hop][...], preferred_element_type=f32)
                     + bs[hop][...])
    nbr = g_nbr[...].reshape(MB, N_NEI, DIM)
    nrr = nr_ref[...]
    ohn = (nrr[:, :, None] ==
           lax.broadcasted_iota(jnp.int32, (MB, N_NEI, N_REL), 2)).astype(f32)
    nrel = jnp.dot(ohn.reshape(MB * N_NEI, N_REL), rtg_ref[...],
                   preferred_element_type=f32).reshape(MB, N_NEI, DIM)
    scores = jnp.sum(v[:, None, :] * nrel, axis=2)
    scores = scores - jnp.max(scores, axis=1, keepdims=True)
    es = jnp.exp(scores)
    w = es / jnp.sum(es, axis=1, keepdims=True)
    agg = jnp.sum(w[:, :, None] * nbr, axis=1)
    cur = jnp.maximum(
        jnp.dot(v0 + agg, Wg0_ref[...], preferred_element_type=f32) + bg0_ref[...],
        0.0)
    logits = jnp.sum(v * cur, axis=1)
    preds_ref[0, 0, :] = 1.0 / (1.0 + jnp.exp(-logits))

    @pl.when(i == 0)
    def _():
        kge_ref[...] = jnp.zeros((1, 1), f32)

    kge_ref[...] += (kge_acc / f32(B * N_MEM)).reshape(1, 1)


def _dense_part(g, mr, item_nr, relation_table, relation_table_gcn,
                W_t0, b_t0, W_t1, b_t1, W_g0, b_g0, interpret=False):
    f32 = jnp.float32
    R3mat = relation_table.reshape(N_REL, DIM, DIM).transpose(1, 0, 2).reshape(
        DIM, N_REL * DIM)
    Rmean = jnp.mean(relation_table.reshape(N_REL, DIM, DIM), axis=2)
    h0b = N_MEMIDX // MB
    seg = SEG_ITEMS // (MB * N_MEM)
    nblk = N_MEMIDX // (MB * N_MEM)
    spec_it = pl.BlockSpec((MB, DIM), lambda i: (i, 0))
    spec_h0 = pl.BlockSpec((MB * N_MEM, DIM), lambda i: (seg + i, 0))
    spec_h1 = pl.BlockSpec((MB * N_MEM, DIM), lambda i: (seg + nblk + i, 0))
    spec_t0 = pl.BlockSpec((MB * N_MEM, DIM), lambda i: (seg + 2 * nblk + i, 0))
    spec_t1 = pl.BlockSpec((MB * N_MEM, DIM), lambda i: (seg + 3 * nblk + i, 0))
    nbr_start = N_TOT - N_NBR
    spec_nbr = pl.BlockSpec((MB * N_NEI, DIM),
                            lambda i: (nbr_start // (MB * N_NEI) + i, 0))
    spec_mr = pl.BlockSpec((N_HOP, MB, N_MEM), lambda i: (0, i, 0))
    spec_nr = pl.BlockSpec((MB, N_NEI), lambda i: (i, 0))
    full = lambda shape: pl.BlockSpec(shape, lambda i: tuple(0 for _ in shape))
    preds2d, kge = pl.pallas_call(
        _tc_body,
        grid=(GRID,),
        in_specs=[spec_it, spec_h0, spec_h1, spec_t0, spec_t1, spec_nbr,
                  spec_mr, spec_nr,
                  full((DIM, N_REL * DIM)), full((N_REL, DIM)),
                  full((N_REL, DIM)),
                  full((DIM, DIM)), full((DIM, DIM)), full((DIM, DIM)),
                  full((1, DIM)), full((1, DIM)), full((1, DIM))],
        out_specs=[pl.BlockSpec((1, 1, MB), lambda i: (i, 0, 0)),
                   pl.BlockSpec((1, 1), lambda i: (0, 0))],
        out_shape=[jax.ShapeDtypeStruct((GRID, 1, MB), f32),
                   jax.ShapeDtypeStruct((1, 1), f32)],
        interpret=interpret,
    )(g, g, g, g, g, g, mr, item_nr,
      R3mat, Rmean, relation_table_gcn,
      W_t0, W_t1, W_g0,
      b_t0.reshape(1, DIM), b_t1.reshape(1, DIM), b_g0.reshape(1, DIM))
    return preds2d.reshape(B), kge[0, 0]


def kernel(items, memories_h, memories_r, memories_t, neighbor_entities,
           neighbor_relations, entity_table, relation_table,
           relation_table_gcn, W_t0, b_t0, W_t1, b_t1, W_g0, b_g0):
    i32 = jnp.int32
    items = items.astype(i32)
    mh = memories_h.astype(i32)
    mr = memories_r.astype(i32)
    mt = memories_t.astype(i32)
    ne = neighbor_entities.astype(i32)
    nr = neighbor_relations.astype(i32)

    item_ne, item_nr = _nbr_gather(ne, nr, items)

    pad = jnp.zeros((SEG_ITEMS - B,), i32)
    idx_all = jnp.concatenate([
        items, pad,
        mh[0].reshape(-1), mh[1].reshape(-1),
        mt[0].reshape(-1), mt[1].reshape(-1),
        item_ne.reshape(-1),
    ])
    g = _entity_gather(entity_table, idx_all)

    return _dense_part(g, mr, item_nr, relation_table, relation_table_gcn,
                       W_t0, b_t0, W_t1, b_t1, W_g0, b_g0)


# speedup vs baseline: 2.9184x; 2.1008x over previous
import functools

import jax
import jax.numpy as jnp
from jax import lax
from jax.experimental import pallas as pl
from jax.experimental.pallas import tpu as pltpu
from jax.experimental.pallas import tpu_sc as plsc

B = 1024
DIM = 32
N_MEM = 32
N_HOP = 2
N_NEI = 16
N_REL = 32

NC, NS = 2, 16
NW = NC * NS
CHUNK = 128

MB = 128
GRID = B // MB

SEG_ITEMS = 4096
N_MEMIDX = B * N_MEM
N_NBR = B * N_NEI
N_TOT = SEG_ITEMS + 2 * N_HOP * N_MEMIDX + N_NBR

def _sc_mesh():
    return plsc.VectorSubcoreMesh(core_axis_name="c", subcore_axis_name="s",
                                  num_cores=NC, num_subcores=NS)


def _nbr_gather(ne, nr, items):
    per_w = B // NW

    @functools.partial(
        pl.kernel,
        out_type=(jax.ShapeDtypeStruct((B, N_NEI), jnp.int32),
                  jax.ShapeDtypeStruct((B, N_NEI), jnp.int32)),
        mesh=_sc_mesh(),
        compiler_params=pltpu.CompilerParams(use_tc_tiling_on_sc=False),
        scratch_types=[pltpu.VMEM((per_w,), jnp.int32),
                       pltpu.VMEM((per_w, N_NEI), jnp.int32),
                       pltpu.SemaphoreType.DMA],
    )
    def k(ne_hbm, nr_hbm, items_hbm, ne_out, nr_out, idx_v, rows_v, sem):
        wid = lax.axis_index("s") * NC + lax.axis_index("c")
        base = wid * per_w
        pltpu.sync_copy(items_hbm.at[pl.ds(base, per_w)], idx_v)
        pltpu.async_copy(ne_hbm.at[idx_v], rows_v, sem).wait()
        pltpu.sync_copy(rows_v, ne_out.at[pl.ds(base, per_w), :])
        pltpu.async_copy(nr_hbm.at[idx_v], rows_v, sem).wait()
        pltpu.sync_copy(rows_v, nr_out.at[pl.ds(base, per_w), :])

    return k(ne, nr, items)


def _entity_gather(table, idx_all):
    n = idx_all.shape[0]
    per_w = n // NW
    n_chunks = per_w // CHUNK

    @functools.partial(
        pl.kernel,
        out_type=jax.ShapeDtypeStruct((n, DIM), jnp.float32),
        mesh=_sc_mesh(),
        compiler_params=pltpu.CompilerParams(use_tc_tiling_on_sc=False),
        scratch_types=[pltpu.VMEM((CHUNK,), jnp.int32),
                       pltpu.VMEM((CHUNK, DIM), jnp.float32),
                       pltpu.SemaphoreType.DMA],
    )
    def k(table_hbm, idx_hbm, out_hbm, idx_v, rows_v, sem):
        wid = lax.axis_index("s") * NC + lax.axis_index("c")
        base = wid * per_w

        def body(c, carry):
            off = base + c * CHUNK
            pltpu.sync_copy(idx_hbm.at[pl.ds(off, CHUNK)], idx_v)
            pltpu.async_copy(table_hbm.at[idx_v], rows_v, sem).wait()
            pltpu.sync_copy(rows_v, out_hbm.at[pl.ds(off, CHUNK), :])
            return carry

        lax.fori_loop(0, n_chunks, body, 0)

    return k(table, idx_all)


def _tc_body(g_it, g_h0, g_h1, g_t0, g_t1, g_nbr, mr_ref, nr_ref,
             R3_ref, Rm_ref, rtg_ref, Wt0_ref, Wt1_ref, Wg0_ref,
             bt0_ref, bt1_ref, bg0_ref, preds_ref, kge_ref):
    i = pl.program_id(0)
    f32 = jnp.float32
    v0 = g_it[...]
    v = v0
    R3 = R3_ref[...]
    Rmean = Rm_ref[...]
    kge_acc = f32(0.0)
    hs = (g_h0, g_h1)
    ts = (g_t0, g_t1)
    Ws = (Wt0_ref, Wt1_ref)
    bs = (bt0_ref, bt1_ref)
    for hop in range(N_HOP):
        h = hs[hop][...].reshape(MB, N_MEM, DIM)
        t = ts[hop][...].reshape(MB, N_MEM, DIM)
        r = mr_ref[hop]
        u = jnp.dot(v, R3, preferred_element_type=f32)
        u3 = u.reshape(MB, N_REL, DIM)
        s = lax.dot_general(h, u3, (((2,), (2,)), ((0,), (0,))),
                            preferred_element_type=f32)
        oh3 = (r[:, :, None] ==
               lax.broadcasted_iota(jnp.int32, (MB, N_MEM, N_REL), 2)).astype(f32)
        att_s = jnp.sum(s * oh3, axis=2)
        att_s = att_s - jnp.max(att_s, axis=1, keepdims=True)
        e = jnp.exp(att_s)
        att = e / jnp.sum(e, axis=1, keepdims=True)
        o = jnp.sum(att[:, :, None] * t, axis=1)
        Rmsel = jnp.dot(oh3.reshape(MB * N_MEM, N_REL), Rmean,
                        preferred_element_type=f32).reshape(MB, N_MEM, DIM)
        diff = h - t + Rmsel
        kge_acc = kge_acc + jnp.sum(diff * diff)
        v = jnp.tanh(jnp.dot(o + v, Ws[hop][...], preferred_element_type=f32)
                     + bs[hop][...])
    nbr = g_nbr[...].reshape(MB, N_NEI, DIM)
    nrr = nr_ref[...]
    ohn = (nrr[:, :, None] ==
           lax.broadcasted_iota(jnp.int32, (MB, N_NEI, N_REL), 2)).astype(f32)
    nrel = jnp.dot(ohn.reshape(MB * N_NEI, N_REL), rtg_ref[...],
                   preferred_element_type=f32).reshape(MB, N_NEI, DIM)
    scores = jnp.sum(v[:, None, :] * nrel, axis=2)
    scores = scores - jnp.max(scores, axis=1, keepdims=True)
    es = jnp.exp(scores)
    w = es / jnp.sum(es, axis=1, keepdims=True)
    agg = jnp.sum(w[:, :, None] * nbr, axis=1)
    cur = jnp.maximum(
        jnp.dot(v0 + agg, Wg0_ref[...], preferred_element_type=f32) + bg0_ref[...],
        0.0)
    logits = jnp.sum(v * cur, axis=1)
    preds_ref[0, 0, :] = 1.0 / (1.0 + jnp.exp(-logits))

    @pl.when(i == 0)
    def _():
        kge_ref[...] = jnp.zeros((1, 1), f32)

    kge_ref[...] += (kge_acc / f32(B * N_MEM)).reshape(1, 1)


def _dense_part(g, mr, item_nr, relation_table, relation_table_gcn,
                W_t0, b_t0, W_t1, b_t1, W_g0, b_g0, interpret=False):
    f32 = jnp.float32
    R3mat = relation_table.reshape(N_REL, DIM, DIM).transpose(1, 0, 2).reshape(
        DIM, N_REL * DIM)
    Rmean = jnp.mean(relation_table.reshape(N_REL, DIM, DIM), axis=2)
    h0b = N_MEMIDX // MB
    seg = SEG_ITEMS // (MB * N_MEM)
    nblk = N_MEMIDX // (MB * N_MEM)
    spec_it = pl.BlockSpec((MB, DIM), lambda i: (i, 0))
    spec_h0 = pl.BlockSpec((MB * N_MEM, DIM), lambda i: (seg + i, 0))
    spec_h1 = pl.BlockSpec((MB * N_MEM, DIM), lambda i: (seg + nblk + i, 0))
    spec_t0 = pl.BlockSpec((MB * N_MEM, DIM), lambda i: (seg + 2 * nblk + i, 0))
    spec_t1 = pl.BlockSpec((MB * N_MEM, DIM), lambda i: (seg + 3 * nblk + i, 0))
    nbr_start = N_TOT - N_NBR
    spec_nbr = pl.BlockSpec((MB * N_NEI, DIM),
                            lambda i: (nbr_start // (MB * N_NEI) + i, 0))
    spec_mr = pl.BlockSpec((N_HOP, MB, N_MEM), lambda i: (0, i, 0))
    spec_nr = pl.BlockSpec((MB, N_NEI), lambda i: (i, 0))
    full = lambda shape: pl.BlockSpec(shape, lambda i: tuple(0 for _ in shape))
    preds2d, kge = pl.pallas_call(
        _tc_body,
        grid=(GRID,),
        in_specs=[spec_it, spec_h0, spec_h1, spec_t0, spec_t1, spec_nbr,
                  spec_mr, spec_nr,
                  full((DIM, N_REL * DIM)), full((N_REL, DIM)),
                  full((N_REL, DIM)),
                  full((DIM, DIM)), full((DIM, DIM)), full((DIM, DIM)),
                  full((1, DIM)), full((1, DIM)), full((1, DIM))],
        out_specs=[pl.BlockSpec((1, 1, MB), lambda i: (i, 0, 0)),
                   pl.BlockSpec((1, 1), lambda i: (0, 0))],
        out_shape=[jax.ShapeDtypeStruct((GRID, 1, MB), f32),
                   jax.ShapeDtypeStruct((1, 1), f32)],
        interpret=interpret,
    )(g, g, g, g, g, g, mr, item_nr,
      R3mat, Rmean, relation_table_gcn,
      W_t0, W_t1, W_g0,
      b_t0.reshape(1, DIM), b_t1.reshape(1, DIM), b_g0.reshape(1, DIM))
    return preds2d.reshape(B), kge[0, 0]


def kernel(items, memories_h, memories_r, memories_t, neighbor_entities,
           neighbor_relations, entity_table, relation_table,
           relation_table_gcn, W_t0, b_t0, W_t1, b_t1, W_g0, b_g0):
    i32 = jnp.int32
    items = items.astype(i32)
    mh = memories_h.astype(i32)
    mr = memories_r.astype(i32)
    mt = memories_t.astype(i32)
    ne = neighbor_entities.astype(i32)
    nr = neighbor_relations.astype(i32)

    item_ne, item_nr = _nbr_gather(ne, nr, items)

    pad = jnp.zeros((SEG_ITEMS - B,), i32)
    idx_all = jnp.concatenate([
        items, pad,
        mh[0].reshape(-1), mh[1].reshape(-1),
        mt[0].reshape(-1), mt[1].reshape(-1),
        item_ne.reshape(-1),
    ])
    g = _entity_gather(entity_table, idx_all)

    return _dense_part(g, mr, item_nr, relation_table, relation_table_gcn,
                       W_t0, b_t0, W_t1, b_t1, W_g0, b_g0)


# speedup vs baseline: 2.9190x; 1.0002x over previous
import functools

import jax
import jax.numpy as jnp
from jax import lax
from jax.experimental import pallas as pl
from jax.experimental.pallas import tpu as pltpu
from jax.experimental.pallas import tpu_sc as plsc

B = 1024
DIM = 32
N_MEM = 32
N_HOP = 2
N_NEI = 16
N_REL = 32

NC, NS = 2, 16
NW = NC * NS
CHUNK = 128

MB = 128
GRID = B // MB

SEG_ITEMS = 4096
N_MEMIDX = B * N_MEM
N_NBR = B * N_NEI
N_TOT = SEG_ITEMS + 2 * N_HOP * N_MEMIDX + N_NBR

def _sc_mesh():
    return plsc.VectorSubcoreMesh(core_axis_name="c", subcore_axis_name="s",
                                  num_cores=NC, num_subcores=NS)


def _nbr_gather(ne, nr, items):
    per_w = B // NW

    @functools.partial(
        pl.kernel,
        out_type=(jax.ShapeDtypeStruct((B, N_NEI), jnp.int32),
                  jax.ShapeDtypeStruct((B, N_NEI), jnp.int32)),
        mesh=_sc_mesh(),
        compiler_params=pltpu.CompilerParams(use_tc_tiling_on_sc=False,
                                             skip_device_barrier=True),
        scratch_types=[pltpu.VMEM((per_w,), jnp.int32),
                       pltpu.VMEM((per_w, N_NEI), jnp.int32),
                       pltpu.SemaphoreType.DMA],
    )
    def k(ne_hbm, nr_hbm, items_hbm, ne_out, nr_out, idx_v, rows_v, sem):
        wid = lax.axis_index("s") * NC + lax.axis_index("c")
        base = wid * per_w
        pltpu.sync_copy(items_hbm.at[pl.ds(base, per_w)], idx_v)
        pltpu.async_copy(ne_hbm.at[idx_v], rows_v, sem).wait()
        pltpu.sync_copy(rows_v, ne_out.at[pl.ds(base, per_w), :])
        pltpu.async_copy(nr_hbm.at[idx_v], rows_v, sem).wait()
        pltpu.sync_copy(rows_v, nr_out.at[pl.ds(base, per_w), :])

    return k(ne, nr, items)


def _entity_gather(table, idx_all):
    n = idx_all.shape[0]
    per_w = n // NW
    n_chunks = per_w // CHUNK

    @functools.partial(
        pl.kernel,
        out_type=jax.ShapeDtypeStruct((n, DIM), jnp.float32),
        mesh=_sc_mesh(),
        compiler_params=pltpu.CompilerParams(use_tc_tiling_on_sc=False,
                                             skip_device_barrier=True),
        scratch_types=[pltpu.VMEM((CHUNK,), jnp.int32),
                       pltpu.VMEM((CHUNK, DIM), jnp.float32),
                       pltpu.SemaphoreType.DMA],
    )
    def k(table_hbm, idx_hbm, out_hbm, idx_v, rows_v, sem):
        wid = lax.axis_index("s") * NC + lax.axis_index("c")
        base = wid * per_w

        def body(c, carry):
            off = base + c * CHUNK
            pltpu.sync_copy(idx_hbm.at[pl.ds(off, CHUNK)], idx_v)
            pltpu.async_copy(table_hbm.at[idx_v], rows_v, sem).wait()
            pltpu.sync_copy(rows_v, out_hbm.at[pl.ds(off, CHUNK), :])
            return carry

        lax.fori_loop(0, n_chunks, body, 0)

    return k(table, idx_all)


def _tc_body(g_it, g_h0, g_h1, g_t0, g_t1, g_nbr, mr_ref, nr_ref,
             R3_ref, Rm_ref, rtg_ref, Wt0_ref, Wt1_ref, Wg0_ref,
             bt0_ref, bt1_ref, bg0_ref, preds_ref, kge_ref):
    i = pl.program_id(0)
    f32 = jnp.float32
    v0 = g_it[...]
    v = v0
    R3 = R3_ref[...]
    Rmean = Rm_ref[...]
    kge_acc = f32(0.0)
    hs = (g_h0, g_h1)
    ts = (g_t0, g_t1)
    Ws = (Wt0_ref, Wt1_ref)
    bs = (bt0_ref, bt1_ref)
    for hop in range(N_HOP):
        h = hs[hop][...].reshape(MB, N_MEM, DIM)
        t = ts[hop][...].reshape(MB, N_MEM, DIM)
        r = mr_ref[hop]
        u = jnp.dot(v, R3, preferred_element_type=f32)
        u3 = u.reshape(MB, N_REL, DIM)
        s = lax.dot_general(h, u3, (((2,), (2,)), ((0,), (0,))),
                            preferred_element_type=f32)
        oh3 = (r[:, :, None] ==
               lax.broadcasted_iota(jnp.int32, (MB, N_MEM, N_REL), 2)).astype(f32)
        att_s = jnp.sum(s * oh3, axis=2)
        att_s = att_s - jnp.max(att_s, axis=1, keepdims=True)
        e = jnp.exp(att_s)
        att = e / jnp.sum(e, axis=1, keepdims=True)
        o = jnp.sum(att[:, :, None] * t, axis=1)
        Rmsel = jnp.dot(oh3.reshape(MB * N_MEM, N_REL), Rmean,
                        preferred_element_type=f32).reshape(MB, N_MEM, DIM)
        diff = h - t + Rmsel
        kge_acc = kge_acc + jnp.sum(diff * diff)
        v = jnp.tanh(jnp.dot(o + v, Ws[hop][...], preferred_element_type=f32)
                     + bs[hop][...])
    nbr = g_nbr[...].reshape(MB, N_NEI, DIM)
    nrr = nr_ref[...]
    ohn = (nrr[:, :, None] ==
           lax.broadcasted_iota(jnp.int32, (MB, N_NEI, N_REL), 2)).astype(f32)
    nrel = jnp.dot(ohn.reshape(MB * N_NEI, N_REL), rtg_ref[...],
                   preferred_element_type=f32).reshape(MB, N_NEI, DIM)
    scores = jnp.sum(v[:, None, :] * nrel, axis=2)
    scores = scores - jnp.max(scores, axis=1, keepdims=True)
    es = jnp.exp(scores)
    w = es / jnp.sum(es, axis=1, keepdims=True)
    agg = jnp.sum(w[:, :, None] * nbr, axis=1)
    cur = jnp.maximum(
        jnp.dot(v0 + agg, Wg0_ref[...], preferred_element_type=f32) + bg0_ref[...],
        0.0)
    logits = jnp.sum(v * cur, axis=1)
    preds_ref[0, 0, :] = 1.0 / (1.0 + jnp.exp(-logits))

    @pl.when(i == 0)
    def _():
        kge_ref[...] = jnp.zeros((1, 1), f32)

    kge_ref[...] += (kge_acc / f32(B * N_MEM)).reshape(1, 1)


def _dense_part(g, mr, item_nr, relation_table, relation_table_gcn,
                W_t0, b_t0, W_t1, b_t1, W_g0, b_g0, interpret=False):
    f32 = jnp.float32
    R3mat = relation_table.reshape(N_REL, DIM, DIM).transpose(1, 0, 2).reshape(
        DIM, N_REL * DIM)
    Rmean = jnp.mean(relation_table.reshape(N_REL, DIM, DIM), axis=2)
    h0b = N_MEMIDX // MB
    seg = SEG_ITEMS // (MB * N_MEM)
    nblk = N_MEMIDX // (MB * N_MEM)
    spec_it = pl.BlockSpec((MB, DIM), lambda i: (i, 0))
    spec_h0 = pl.BlockSpec((MB * N_MEM, DIM), lambda i: (seg + i, 0))
    spec_h1 = pl.BlockSpec((MB * N_MEM, DIM), lambda i: (seg + nblk + i, 0))
    spec_t0 = pl.BlockSpec((MB * N_MEM, DIM), lambda i: (seg + 2 * nblk + i, 0))
    spec_t1 = pl.BlockSpec((MB * N_MEM, DIM), lambda i: (seg + 3 * nblk + i, 0))
    nbr_start = N_TOT - N_NBR
    spec_nbr = pl.BlockSpec((MB * N_NEI, DIM),
                            lambda i: (nbr_start // (MB * N_NEI) + i, 0))
    spec_mr = pl.BlockSpec((N_HOP, MB, N_MEM), lambda i: (0, i, 0))
    spec_nr = pl.BlockSpec((MB, N_NEI), lambda i: (i, 0))
    full = lambda shape: pl.BlockSpec(shape, lambda i: tuple(0 for _ in shape))
    preds2d, kge = pl.pallas_call(
        _tc_body,
        grid=(GRID,),
        in_specs=[spec_it, spec_h0, spec_h1, spec_t0, spec_t1, spec_nbr,
                  spec_mr, spec_nr,
                  full((DIM, N_REL * DIM)), full((N_REL, DIM)),
                  full((N_REL, DIM)),
                  full((DIM, DIM)), full((DIM, DIM)), full((DIM, DIM)),
                  full((1, DIM)), full((1, DIM)), full((1, DIM))],
        out_specs=[pl.BlockSpec((1, 1, MB), lambda i: (i, 0, 0)),
                   pl.BlockSpec((1, 1), lambda i: (0, 0))],
        out_shape=[jax.ShapeDtypeStruct((GRID, 1, MB), f32),
                   jax.ShapeDtypeStruct((1, 1), f32)],
        interpret=interpret,
    )(g, g, g, g, g, g, mr, item_nr,
      R3mat, Rmean, relation_table_gcn,
      W_t0, W_t1, W_g0,
      b_t0.reshape(1, DIM), b_t1.reshape(1, DIM), b_g0.reshape(1, DIM))
    return preds2d.reshape(B), kge[0, 0]


def kernel(items, memories_h, memories_r, memories_t, neighbor_entities,
           neighbor_relations, entity_table, relation_table,
           relation_table_gcn, W_t0, b_t0, W_t1, b_t1, W_g0, b_g0):
    i32 = jnp.int32
    items = items.astype(i32)
    mh = memories_h.astype(i32)
    mr = memories_r.astype(i32)
    mt = memories_t.astype(i32)
    ne = neighbor_entities.astype(i32)
    nr = neighbor_relations.astype(i32)

    item_ne, item_nr = _nbr_gather(ne, nr, items)

    pad = jnp.zeros((SEG_ITEMS - B,), i32)
    idx_all = jnp.concatenate([
        items, pad,
        mh[0].reshape(-1), mh[1].reshape(-1),
        mt[0].reshape(-1), mt[1].reshape(-1),
        item_ne.reshape(-1),
    ])
    g = _entity_gather(entity_table, idx_all)

    return _dense_part(g, mr, item_nr, relation_table, relation_table_gcn,
                       W_t0, b_t0, W_t1, b_t1, W_g0, b_g0)


# speedup vs baseline: 3.2210x; 1.1035x over previous
import functools

import jax
import jax.numpy as jnp
from jax import lax
from jax.experimental import pallas as pl
from jax.experimental.pallas import tpu as pltpu
from jax.experimental.pallas import tpu_sc as plsc

B = 1024
DIM = 32
N_MEM = 32
N_HOP = 2
N_NEI = 16
N_REL = 32

NC, NS = 2, 16
NW = NC * NS
CHUNK = 128

MB = 128
GRID = B // MB

SEG_ITEMS = 4096
N_MEMIDX = B * N_MEM
N_NBR = B * N_NEI
N_TOT = SEG_ITEMS + 2 * N_HOP * N_MEMIDX + N_NBR

def _sc_mesh():
    return plsc.VectorSubcoreMesh(core_axis_name="c", subcore_axis_name="s",
                                  num_cores=NC, num_subcores=NS)


def _nbr_gather(ne, nr, items):
    per_w = B // NW

    @functools.partial(
        pl.kernel,
        out_type=(jax.ShapeDtypeStruct((B, N_NEI), jnp.int32),
                  jax.ShapeDtypeStruct((B, N_NEI), jnp.int32)),
        mesh=_sc_mesh(),
        compiler_params=pltpu.CompilerParams(use_tc_tiling_on_sc=False,
                                             skip_device_barrier=True),
        scratch_types=[pltpu.VMEM((per_w,), jnp.int32),
                       pltpu.VMEM((per_w, N_NEI), jnp.int32),
                       pltpu.SemaphoreType.DMA],
    )
    def k(ne_hbm, nr_hbm, items_hbm, ne_out, nr_out, idx_v, rows_v, sem):
        wid = lax.axis_index("s") * NC + lax.axis_index("c")
        base = wid * per_w
        pltpu.sync_copy(items_hbm.at[pl.ds(base, per_w)], idx_v)
        pltpu.async_copy(ne_hbm.at[idx_v], rows_v, sem).wait()
        pltpu.sync_copy(rows_v, ne_out.at[pl.ds(base, per_w), :])
        pltpu.async_copy(nr_hbm.at[idx_v], rows_v, sem).wait()
        pltpu.sync_copy(rows_v, nr_out.at[pl.ds(base, per_w), :])

    return k(ne, nr, items)


def _entity_gather(table, idx_all):
    n = idx_all.shape[0]
    per_w = n // NW
    n_chunks = 2
    chunk = per_w // n_chunks

    @functools.partial(
        pl.kernel,
        out_type=jax.ShapeDtypeStruct((n, DIM), jnp.float32),
        mesh=_sc_mesh(),
        compiler_params=pltpu.CompilerParams(use_tc_tiling_on_sc=False,
                                             skip_device_barrier=True),
        scratch_types=[pltpu.VMEM((per_w,), jnp.int32),
                       pltpu.VMEM((chunk, DIM), jnp.float32),
                       pltpu.SemaphoreType.DMA],
    )
    def k(table_hbm, idx_hbm, out_hbm, idx_v, rows_v, sem):
        wid = lax.axis_index("s") * NC + lax.axis_index("c")
        base = wid * per_w
        pltpu.sync_copy(idx_hbm.at[pl.ds(base, per_w)], idx_v)
        for c in range(n_chunks):
            off = base + c * chunk
            pltpu.async_copy(table_hbm.at[idx_v.at[pl.ds(c * chunk, chunk)]],
                             rows_v, sem).wait()
            pltpu.sync_copy(rows_v, out_hbm.at[pl.ds(off, chunk), :])

    return k(table, idx_all)


def _tc_body(g_it, g_h0, g_h1, g_t0, g_t1, g_nbr, mr_ref, nr_ref,
             R3_ref, Rm_ref, rtg_ref, Wt0_ref, Wt1_ref, Wg0_ref,
             bt0_ref, bt1_ref, bg0_ref, preds_ref, kge_ref):
    i = pl.program_id(0)
    f32 = jnp.float32
    v0 = g_it[...]
    v = v0
    R3 = R3_ref[...]
    Rmean = Rm_ref[...]
    kge_acc = f32(0.0)
    hs = (g_h0, g_h1)
    ts = (g_t0, g_t1)
    Ws = (Wt0_ref, Wt1_ref)
    bs = (bt0_ref, bt1_ref)
    for hop in range(N_HOP):
        h = hs[hop][...].reshape(MB, N_MEM, DIM)
        t = ts[hop][...].reshape(MB, N_MEM, DIM)
        r = mr_ref[hop]
        u = jnp.dot(v, R3, preferred_element_type=f32)
        u3 = u.reshape(MB, N_REL, DIM)
        s = lax.dot_general(h, u3, (((2,), (2,)), ((0,), (0,))),
                            preferred_element_type=f32)
        oh3 = (r[:, :, None] ==
               lax.broadcasted_iota(jnp.int32, (MB, N_MEM, N_REL), 2)).astype(f32)
        att_s = jnp.sum(s * oh3, axis=2)
        att_s = att_s - jnp.max(att_s, axis=1, keepdims=True)
        e = jnp.exp(att_s)
        att = e / jnp.sum(e, axis=1, keepdims=True)
        o = jnp.sum(att[:, :, None] * t, axis=1)
        Rmsel = jnp.dot(oh3.reshape(MB * N_MEM, N_REL), Rmean,
                        preferred_element_type=f32).reshape(MB, N_MEM, DIM)
        diff = h - t + Rmsel
        kge_acc = kge_acc + jnp.sum(diff * diff)
        v = jnp.tanh(jnp.dot(o + v, Ws[hop][...], preferred_element_type=f32)
                     + bs[hop][...])
    nbr = g_nbr[...].reshape(MB, N_NEI, DIM)
    nrr = nr_ref[...]
    ohn = (nrr[:, :, None] ==
           lax.broadcasted_iota(jnp.int32, (MB, N_NEI, N_REL), 2)).astype(f32)
    nrel = jnp.dot(ohn.reshape(MB * N_NEI, N_REL), rtg_ref[...],
                   preferred_element_type=f32).reshape(MB, N_NEI, DIM)
    scores = jnp.sum(v[:, None, :] * nrel, axis=2)
    scores = scores - jnp.max(scores, axis=1, keepdims=True)
    es = jnp.exp(scores)
    w = es / jnp.sum(es, axis=1, keepdims=True)
    agg = jnp.sum(w[:, :, None] * nbr, axis=1)
    cur = jnp.maximum(
        jnp.dot(v0 + agg, Wg0_ref[...], preferred_element_type=f32) + bg0_ref[...],
        0.0)
    logits = jnp.sum(v * cur, axis=1)
    preds_ref[0, 0, :] = 1.0 / (1.0 + jnp.exp(-logits))

    @pl.when(i == 0)
    def _():
        kge_ref[...] = jnp.zeros((1, 1), f32)

    kge_ref[...] += (kge_acc / f32(B * N_MEM)).reshape(1, 1)


def _dense_part(g, mr, item_nr, relation_table, relation_table_gcn,
                W_t0, b_t0, W_t1, b_t1, W_g0, b_g0, interpret=False):
    f32 = jnp.float32
    R3mat = relation_table.reshape(N_REL, DIM, DIM).transpose(1, 0, 2).reshape(
        DIM, N_REL * DIM)
    Rmean = jnp.mean(relation_table.reshape(N_REL, DIM, DIM), axis=2)
    h0b = N_MEMIDX // MB
    seg = SEG_ITEMS // (MB * N_MEM)
    nblk = N_MEMIDX // (MB * N_MEM)
    spec_it = pl.BlockSpec((MB, DIM), lambda i: (i, 0))
    spec_h0 = pl.BlockSpec((MB * N_MEM, DIM), lambda i: (seg + i, 0))
    spec_h1 = pl.BlockSpec((MB * N_MEM, DIM), lambda i: (seg + nblk + i, 0))
    spec_t0 = pl.BlockSpec((MB * N_MEM, DIM), lambda i: (seg + 2 * nblk + i, 0))
    spec_t1 = pl.BlockSpec((MB * N_MEM, DIM), lambda i: (seg + 3 * nblk + i, 0))
    nbr_start = N_TOT - N_NBR
    spec_nbr = pl.BlockSpec((MB * N_NEI, DIM),
                            lambda i: (nbr_start // (MB * N_NEI) + i, 0))
    spec_mr = pl.BlockSpec((N_HOP, MB, N_MEM), lambda i: (0, i, 0))
    spec_nr = pl.BlockSpec((MB, N_NEI), lambda i: (i, 0))
    full = lambda shape: pl.BlockSpec(shape, lambda i: tuple(0 for _ in shape))
    preds2d, kge = pl.pallas_call(
        _tc_body,
        grid=(GRID,),
        in_specs=[spec_it, spec_h0, spec_h1, spec_t0, spec_t1, spec_nbr,
                  spec_mr, spec_nr,
                  full((DIM, N_REL * DIM)), full((N_REL, DIM)),
                  full((N_REL, DIM)),
                  full((DIM, DIM)), full((DIM, DIM)), full((DIM, DIM)),
                  full((1, DIM)), full((1, DIM)), full((1, DIM))],
        out_specs=[pl.BlockSpec((1, 1, MB), lambda i: (i, 0, 0)),
                   pl.BlockSpec((1, 1), lambda i: (0, 0))],
        out_shape=[jax.ShapeDtypeStruct((GRID, 1, MB), f32),
                   jax.ShapeDtypeStruct((1, 1), f32)],
        interpret=interpret,
    )(g, g, g, g, g, g, mr, item_nr,
      R3mat, Rmean, relation_table_gcn,
      W_t0, W_t1, W_g0,
      b_t0.reshape(1, DIM), b_t1.reshape(1, DIM), b_g0.reshape(1, DIM))
    return preds2d.reshape(B), kge[0, 0]


def kernel(items, memories_h, memories_r, memories_t, neighbor_entities,
           neighbor_relations, entity_table, relation_table,
           relation_table_gcn, W_t0, b_t0, W_t1, b_t1, W_g0, b_g0):
    i32 = jnp.int32
    items = items.astype(i32)
    mh = memories_h.astype(i32)
    mr = memories_r.astype(i32)
    mt = memories_t.astype(i32)
    ne = neighbor_entities.astype(i32)
    nr = neighbor_relations.astype(i32)

    item_ne, item_nr = _nbr_gather(ne, nr, items)

    pad = jnp.zeros((SEG_ITEMS - B,), i32)
    idx_all = jnp.concatenate([
        items, pad,
        mh[0].reshape(-1), mh[1].reshape(-1),
        mt[0].reshape(-1), mt[1].reshape(-1),
        item_ne.reshape(-1),
    ])
    g = _entity_gather(entity_table, idx_all)

    return _dense_part(g, mr, item_nr, relation_table, relation_table_gcn,
                       W_t0, b_t0, W_t1, b_t1, W_g0, b_g0)
